# baseline (speedup 1.0000x reference)
import jax
import jax.numpy as jnp
import numpy as np
from jax.experimental import pallas as pl

_FEAT_STRIDE = 16
_SCALES = np.array([8.0, 16.0, 32.0], dtype=np.float32)
_RATIOS = np.array([0.5, 1.0, 2.0], dtype=np.float32)
_PRE_N = 12000
_POST_N = 2000
_THRESH = 0.7
_MIN_SIZE = 16.0


def _anchors_np(H, W):
    base = float(_FEAT_STRIDE)
    py = px = base / 2.0
    A = len(_RATIOS) * len(_SCALES)
    ab = np.zeros((A, 4), dtype=np.float32)
    for i, r in enumerate(_RATIOS):
        for j, s in enumerate(_SCALES):
            h = base * s * np.sqrt(r)
            w = base * s * np.sqrt(1.0 / r)
            k = i * len(_SCALES) + j
            ab[k] = [py - h / 2.0, px - w / 2.0, py + h / 2.0, px + w / 2.0]
    sy = np.arange(0, H * _FEAT_STRIDE, _FEAT_STRIDE, dtype=np.float32)
    sx = np.arange(0, W * _FEAT_STRIDE, _FEAT_STRIDE, dtype=np.float32)
    gx, gy = np.meshgrid(sx, sy)
    shift = np.stack([gy.ravel(), gx.ravel(), gy.ravel(), gx.ravel()], axis=1)
    return (ab[None, :, :] + shift[:, None, :]).reshape(-1, 4).astype(np.float32)


def _trunk_body(x_ref, w_ref, cb_ref, hw_ref, hb_ref, loc_ref, p0_ref, p1_ref):
    feat = jnp.dot(x_ref[...], w_ref[...], preferred_element_type=jnp.float32)
    feat = jax.nn.relu(feat + cb_ref[...])
    heads = jnp.dot(feat, hw_ref[...], preferred_element_type=jnp.float32) + hb_ref[...]
    loc_ref[...] = heads[:, 0:36]
    l0 = heads[:, 36:45]
    l1 = heads[:, 45:54]
    m = jnp.maximum(l0, l1)
    e0 = jnp.exp(l0 - m)
    e1 = jnp.exp(l1 - m)
    s = e0 + e1
    p0_ref[...] = e0 / s
    p1_ref[...] = e1 / s


def _trunk_pallas(x9, w9, conv_b, head_w, head_b):
    M, K = x9.shape
    TM = 512
    return pl.pallas_call(
        _trunk_body,
        grid=(M // TM,),
        in_specs=[
            pl.BlockSpec((TM, K), lambda i: (i, 0)),
            pl.BlockSpec((K, 512), lambda i: (0, 0)),
            pl.BlockSpec((1, 512), lambda i: (0, 0)),
            pl.BlockSpec((512, 64), lambda i: (0, 0)),
            pl.BlockSpec((1, 64), lambda i: (0, 0)),
        ],
        out_specs=[
            pl.BlockSpec((TM, 36), lambda i: (i, 0)),
            pl.BlockSpec((TM, 9), lambda i: (i, 0)),
            pl.BlockSpec((TM, 9), lambda i: (i, 0)),
        ],
        out_shape=[
            jax.ShapeDtypeStruct((M, 36), jnp.float32),
            jax.ShapeDtypeStruct((M, 9), jnp.float32),
            jax.ShapeDtypeStruct((M, 9), jnp.float32),
        ],
    )(x9, w9, conv_b, head_w, head_b)


def _conv2d(x, w, b, pad):
    out = jax.lax.conv_general_dilated(x, w, (1, 1), [(pad, pad), (pad, pad)],
                                       dimension_numbers=('NCHW', 'OIHW', 'NCHW'))
    return out + b[None, :, None, None]


def _loc2bbox(src, loc):
    sh = src[:, 2] - src[:, 0]
    sw = src[:, 3] - src[:, 1]
    cy = src[:, 0] + 0.5 * sh
    cx = src[:, 1] + 0.5 * sw
    dy, dx, dh, dw = loc[:, 0], loc[:, 1], loc[:, 2], loc[:, 3]
    ncy = dy * sh + cy
    ncx = dx * sw + cx
    nh = jnp.exp(dh) * sh
    nw = jnp.exp(dw) * sw
    return jnp.stack([ncy - 0.5 * nh, ncx - 0.5 * nw, ncy + 0.5 * nh, ncx + 0.5 * nw], axis=1)


def _nms_seq(boxes, scores, thresh, max_out):
    areas = (boxes[:, 2] - boxes[:, 0]) * (boxes[:, 3] - boxes[:, 1])
    def body(i, state):
        s, keep = state
        idx = jnp.argmax(s)
        keep = keep.at[i].set(idx.astype(jnp.int32))
        box = boxes[idx]
        yy1 = jnp.maximum(box[0], boxes[:, 0])
        xx1 = jnp.maximum(box[1], boxes[:, 1])
        yy2 = jnp.minimum(box[2], boxes[:, 2])
        xx2 = jnp.minimum(box[3], boxes[:, 3])
        inter = jnp.maximum(yy2 - yy1, 0.0) * jnp.maximum(xx2 - xx1, 0.0)
        iou = inter / (areas[idx] + areas - inter + 1e-9)
        s = jnp.where(iou > thresh, -jnp.inf, s)
        s = s.at[idx].set(-jnp.inf)
        return s, keep
    keep0 = jnp.zeros((max_out,), dtype=jnp.int32)
    _, keep = jax.lax.fori_loop(0, max_out, body, (scores, keep0))
    return keep


def kernel(inputs, conv_w, conv_b, bf_w, bf_b, loc_w, loc_b, img_size):
    B, C, H, W = inputs.shape
    A = 9
    mid = conv_w.shape[0]
    anchors = jnp.asarray(_anchors_np(H, W))
    img = jnp.asarray(img_size).astype(jnp.float32)

    x = jnp.transpose(inputs[0], (1, 2, 0))
    xp = jnp.pad(x, ((1, 1), (1, 1), (0, 0)))
    x9 = jnp.stack([xp[ky:ky + H, kx:kx + W, :]
                    for ky in range(3) for kx in range(3)], axis=2).reshape(H * W, 9 * C)
    w9 = jnp.transpose(conv_w, (2, 3, 1, 0)).reshape(9 * C, mid)
    locw = jnp.transpose(loc_w.reshape(A * 4, mid), (1, 0))
    bfw = jnp.transpose(bf_w.reshape(A * 2, mid), (1, 0))
    bf0 = bfw[:, 0::2]
    bf1 = bfw[:, 1::2]
    head_w = jnp.concatenate([locw, bf0, bf1, jnp.zeros((mid, 10), jnp.float32)], axis=1)
    head_b = jnp.concatenate([loc_b, bf_b[0::2], bf_b[1::2], jnp.zeros((10,), jnp.float32)])
    loc_flat, p0, p1 = _trunk_pallas(x9, w9, conv_b.reshape(1, mid),
                                     head_w, head_b.reshape(1, 64))
    loc_out = loc_flat.reshape(1, H * W * A, 4)
    bf_out = jnp.stack([p0.reshape(-1), p1.reshape(-1)], axis=1).reshape(1, H * W * A, 2)

    feat_r = jax.nn.relu(_conv2d(inputs, conv_w, conv_b, 1))
    loc_r = _conv2d(feat_r, loc_w, loc_b, 0)
    loc_r = jnp.transpose(loc_r, (0, 2, 3, 1)).reshape(B, -1, 4)
    bf_r = _conv2d(feat_r, bf_w, bf_b, 0)
    bf_r = jnp.transpose(bf_r, (0, 2, 3, 1)).reshape(B, H, W, A, 2)
    bf_r = jax.nn.softmax(bf_r, axis=4)
    fg_r = bf_r[..., 1].reshape(B, -1)

    roi = _loc2bbox(anchors, loc_r[0])
    roi = jnp.stack([jnp.clip(roi[:, 0], 0.0, img),
                     jnp.clip(roi[:, 1], 0.0, img),
                     jnp.clip(roi[:, 2], 0.0, img),
                     jnp.clip(roi[:, 3], 0.0, img)], axis=1)
    hs = roi[:, 2] - roi[:, 0]
    ws = roi[:, 3] - roi[:, 1]
    score = jnp.where((hs >= _MIN_SIZE) & (ws >= _MIN_SIZE), fg_r[0], -jnp.inf)
    pre_n = min(_PRE_N, anchors.shape[0])
    top_scores, order = jax.lax.top_k(jax.lax.stop_gradient(score), pre_n)
    roi_sel = roi[order]
    keep = _nms_seq(jax.lax.stop_gradient(roi_sel), top_scores, _THRESH, _POST_N)
    rois = roi_sel[keep]
    rois_idx = jnp.zeros((_POST_N,), dtype=jnp.int32)
    return rois, rois_idx, anchors, loc_out, bf_out
